# baseline (speedup 1.0000x reference)
import functools

import jax
import jax.numpy as jnp
from jax import lax
from jax.experimental import pallas as pl
from jax.experimental.pallas import tpu as pltpu
from jax.experimental.pallas import tpu_sc as plsc

B = 8
N = 2048
SEGS = 4
SEG = N // SEGS
V_PAD = 1024
L = 16
CHUNKS = SEG // L


def _body(x_hbm, out_hbm, xv, ov, hist, nb, spm, sem):
    c = lax.axis_index("c")
    s = lax.axis_index("s")
    lrow = s // SEGS
    seg = s % SEGS
    row = c * (B // 2) + lrow
    base = row * N + seg * SEG

    in_cp = pltpu.async_copy(x_hbm.at[pl.ds(base, SEG)], xv, sem)

    @plsc.parallel_loop(0, V_PAD // L)
    def _(i):
        hist[pl.ds(i * L, L)] = jnp.zeros((L,), jnp.float32)

    in_cp.wait()

    def chunk(i, _):
        v = xv[pl.ds(i * L, L)]
        prev = plsc.load_gather(hist, [v])
        rank, last = plsc.scan_count(v)
        cnt = prev + rank.astype(jnp.float32)
        ov[pl.ds(i * L, L)] = cnt
        plsc.store_scatter(hist, [v], cnt, mask=last)
        return 0

    lax.fori_loop(0, CHUNKS, chunk, 0)

    @pl.when(seg < SEGS - 1)
    def _():
        pltpu.sync_copy(hist, spm.at[s])

    plsc.subcore_barrier()

    @pl.when(seg > 0)
    def _():
        def fire(k, _):
            pltpu.async_copy(spm.at[s - seg + k], nb.at[pl.ds(k * V_PAD, V_PAD)], sem)
            return 0

        lax.fori_loop(0, seg, fire, 0)

        def drain(k, _):
            pltpu.make_async_copy(
                spm.at[s - seg], nb.at[pl.ds(0, V_PAD)], sem
            ).wait()
            return 0

        lax.fori_loop(0, seg, drain, 0)

        def off(i, _):
            d = pl.ds(i * L, L)
            v = xv[d]

            def gath(k, acc):
                return acc + plsc.load_gather(nb, [v + k * V_PAD])

            ov[d] = lax.fori_loop(0, seg, gath, ov[d])
            return 0

        lax.fori_loop(0, CHUNKS, off, 0)

    pltpu.sync_copy(ov, out_hbm.at[pl.ds(base, SEG)])


@jax.jit
def _counts(x):
    run = pl.kernel(
        _body,
        out_type=jax.ShapeDtypeStruct((B * N,), jnp.float32),
        mesh=plsc.VectorSubcoreMesh(core_axis_name="c", subcore_axis_name="s"),
        scratch_types=[
            pltpu.VMEM((SEG,), jnp.int32),
            pltpu.VMEM((SEG,), jnp.float32),
            pltpu.VMEM((V_PAD,), jnp.float32),
            pltpu.VMEM(((SEGS - 1) * V_PAD,), jnp.float32),
            pltpu.VMEM_SHARED((16, V_PAD), jnp.float32),
            pltpu.SemaphoreType.DMA,
        ],
        compiler_params=pltpu.CompilerParams(needs_layout_passes=False),
    )
    return run(x.astype(jnp.int32).reshape(B * N))


def kernel(x):
    return _counts(x).reshape(B, N, 1)


# speedup vs baseline: 1.0080x; 1.0080x over previous
import functools

import jax
import jax.numpy as jnp
from jax import lax
from jax.experimental import pallas as pl
from jax.experimental.pallas import tpu as pltpu
from jax.experimental.pallas import tpu_sc as plsc

B = 8
N = 2048
SEGS = 4
SEG = N // SEGS
V_PAD = 1024
L = 16
CHUNKS = SEG // L


def _body(x_hbm, out_hbm, xv, ov, hist, nb0, nb1, nb2, spm, sem):
    c = lax.axis_index("c")
    s = lax.axis_index("s")
    lrow = s // SEGS
    seg = s % SEGS
    row = c * (B // 2) + lrow
    base = row * N + seg * SEG

    in_cp = pltpu.async_copy(x_hbm.at[pl.ds(base, SEG)], xv, sem)

    def zero(i, _):
        hist[pl.ds(i * L, L)] = jnp.zeros((L,), jnp.float32)
        return 0

    lax.fori_loop(0, V_PAD // L, zero, 0)
    in_cp.wait()

    def chunk(i, _):
        v = xv[pl.ds(i * L, L)]
        prev = plsc.load_gather(hist, [v])
        rank, last = plsc.scan_count(v)
        cnt = prev + rank.astype(jnp.float32)
        ov[pl.ds(i * L, L)] = cnt
        plsc.store_scatter(hist, [v], cnt, mask=last)
        return 0

    lax.fori_loop(0, CHUNKS, chunk, 0)

    @pl.when(seg < SEGS - 1)
    def _():
        pltpu.sync_copy(hist, spm.at[s])

    plsc.subcore_barrier()

    @pl.when(seg > 0)
    def _():
        nbufs = (nb0, nb1, nb2)
        for k in range(SEGS - 1):
            @pl.when(seg > k)
            def _():
                pltpu.async_copy(spm.at[s - seg + k], nbufs[k], sem)
        for k in range(SEGS - 1):
            @pl.when(seg > k)
            def _():
                pltpu.make_async_copy(spm.at[s - seg + k], nbufs[k], sem).wait()

        def off(i, _):
            d = pl.ds(i * L, L)
            v = xv[d]
            acc = ov[d] + plsc.load_gather(nb0, [v])
            for k in range(1, SEGS - 1):
                gk = plsc.load_gather(nbufs[k], [v])
                acc = acc + jnp.where(seg > k, gk, jnp.zeros((L,), jnp.float32))
            ov[d] = acc
            return 0

        lax.fori_loop(0, CHUNKS, off, 0)

    pltpu.sync_copy(ov, out_hbm.at[pl.ds(base, SEG)])


@jax.jit
def _counts(x):
    run = pl.kernel(
        _body,
        out_type=jax.ShapeDtypeStruct((B * N,), jnp.float32),
        mesh=plsc.VectorSubcoreMesh(core_axis_name="c", subcore_axis_name="s"),
        scratch_types=[
            pltpu.VMEM((SEG,), jnp.int32),
            pltpu.VMEM((SEG,), jnp.float32),
            pltpu.VMEM((V_PAD,), jnp.float32),
            pltpu.VMEM((V_PAD,), jnp.float32),
            pltpu.VMEM((V_PAD,), jnp.float32),
            pltpu.VMEM((V_PAD,), jnp.float32),
            pltpu.VMEM_SHARED((16, V_PAD), jnp.float32),
            pltpu.SemaphoreType.DMA,
        ],
        compiler_params=pltpu.CompilerParams(needs_layout_passes=False),
    )
    return run(x.astype(jnp.int32).reshape(B * N))


def kernel(x):
    return _counts(x).reshape(B, N, 1)


# speedup vs baseline: 1.0648x; 1.0564x over previous
import functools

import jax
import jax.numpy as jnp
from jax import lax
from jax.experimental import pallas as pl
from jax.experimental.pallas import tpu as pltpu
from jax.experimental.pallas import tpu_sc as plsc

B = 8
N = 2048
SEGS = 2
SEG = N // SEGS
V_PAD = 1024
L = 16
CHUNKS = SEG // L


def _body(x_hbm, out_hbm, xv, ov, hist, nb0, spm, sem):
    s = lax.axis_index("s")
    row = s // SEGS
    seg = s % SEGS
    base = row * N + seg * SEG

    in_cp = pltpu.async_copy(x_hbm.at[pl.ds(base, SEG)], xv, sem)

    def zero(i, _):
        hist[pl.ds(i * L, L)] = jnp.zeros((L,), jnp.float32)
        return 0

    lax.fori_loop(0, V_PAD // L, zero, 0)
    in_cp.wait()

    def chunk(i, _):
        v = xv[pl.ds(i * L, L)]
        prev = plsc.load_gather(hist, [v])
        rank, last = plsc.scan_count(v)
        cnt = prev + rank.astype(jnp.float32)
        ov[pl.ds(i * L, L)] = cnt
        plsc.store_scatter(hist, [v], cnt, mask=last)
        return 0

    lax.fori_loop(0, CHUNKS, chunk, 0)

    @pl.when(seg == 0)
    def _():
        pltpu.sync_copy(hist, spm.at[s])

    plsc.subcore_barrier()

    @pl.when(seg > 0)
    def _():
        pltpu.sync_copy(spm.at[s - 1], nb0)

        def off(i, _):
            d = pl.ds(i * L, L)
            ov[d] = ov[d] + plsc.load_gather(nb0, [xv[d]])
            return 0

        lax.fori_loop(0, CHUNKS, off, 0)

    pltpu.sync_copy(ov, out_hbm.at[pl.ds(base, SEG)])


@jax.jit
def _counts(x):
    run = pl.kernel(
        _body,
        out_type=jax.ShapeDtypeStruct((B * N,), jnp.float32),
        mesh=plsc.VectorSubcoreMesh(
            core_axis_name="c", subcore_axis_name="s", num_cores=1
        ),
        scratch_types=[
            pltpu.VMEM((SEG,), jnp.int32),
            pltpu.VMEM((SEG,), jnp.float32),
            pltpu.VMEM((V_PAD,), jnp.float32),
            pltpu.VMEM((V_PAD,), jnp.float32),
            pltpu.VMEM_SHARED((16, V_PAD), jnp.float32),
            pltpu.SemaphoreType.DMA,
        ],
        compiler_params=pltpu.CompilerParams(needs_layout_passes=False),
    )
    return run(x.astype(jnp.int32).reshape(B * N))


def kernel(x):
    return _counts(x).reshape(B, N, 1)
